# baseline (speedup 1.0000x reference)
import jax
import jax.numpy as jnp
from jax.experimental import pallas as pl
from jax.experimental.pallas import tpu as pltpu

VOCAB = 1000
EMBD = 32
DPAD = 128
BATCH = 1024
SEQ = 50
BB = 16


def kernel(idx, tok_table, pos_table, W, b):
  idx32 = idx.astype(jnp.int32)
  tok_pad = jnp.pad(tok_table, ((0, 0), (0, DPAD - EMBD)))
  pos_pad = jnp.pad(pos_table, ((0, 0), (0, DPAD - EMBD)))
  w_pad = jnp.pad(W, ((0, DPAD - EMBD), (0, 0)))

  def head(idx_ref, tok_ref, pos_ref, w_ref, b_ref, out_ref):
    tok = tok_ref[...]
    pos = pos_ref[...]
    w_v = w_ref[...]
    b_v = b_ref[...]
    iota_v = jax.lax.broadcasted_iota(jnp.int32, (SEQ, VOCAB), 1)
    for j in range(BB):
      onehot = (idx_ref[j][:, None] == iota_v).astype(jnp.float32)
      emb = jnp.dot(onehot, tok, preferred_element_type=jnp.float32)
      x = emb + pos
      out_ref[j] = jnp.dot(x, w_v, preferred_element_type=jnp.float32) + b_v

  return pl.pallas_call(
      head,
      grid=(BATCH // BB,),
      in_specs=[
          pl.BlockSpec((BB, SEQ), lambda i: (i, 0)),
          pl.BlockSpec((VOCAB, DPAD), lambda i: (0, 0)),
          pl.BlockSpec((SEQ, DPAD), lambda i: (0, 0)),
          pl.BlockSpec((DPAD, VOCAB), lambda i: (0, 0)),
          pl.BlockSpec((1, VOCAB), lambda i: (0, 0)),
      ],
      out_specs=pl.BlockSpec((BB, SEQ, VOCAB), lambda i: (i, 0, 0)),
      out_shape=jax.ShapeDtypeStruct((BATCH, SEQ, VOCAB), jnp.float32),
      compiler_params=pltpu.CompilerParams(
          dimension_semantics=("arbitrary",),
      ),
  )(idx32, tok_pad, pos_pad, w_pad, b.reshape(1, VOCAB))
